# baseline (speedup 1.0000x reference)
import functools

import jax
import jax.numpy as jnp
from jax import lax
from jax.experimental import pallas as pl
from jax.experimental.pallas import tpu as pltpu
from jax.experimental.pallas import tpu_sc as plsc

NUM_ITEMS = 1000000
EMBED_DIM = 64
BATCH = 16384
HIST = 50

_INFO = plsc.get_sparse_core_info()
_NC, _NS = _INFO.num_cores, _INFO.num_subcores
_NW = _NC * _NS
_BPW = BATCH // _NW
_NBT = _BPW // 128
_NDT = EMBED_DIM // 8


def _sc_gather(idx_t, table):
    mesh = plsc.VectorSubcoreMesh(core_axis_name="c", subcore_axis_name="s")

    @functools.partial(
        pl.kernel,
        out_type=jax.ShapeDtypeStruct(
            (HIST, _NDT, BATCH // 128, 8, 128), jnp.float32),
        mesh=mesh,
        scratch_types=[
            pltpu.VMEM((HIST, _BPW), jnp.int32),
            pltpu.VMEM((_NBT, 128, EMBED_DIM), jnp.float32),
            pltpu.VMEM((2, _NDT, _NBT, 8, 128), jnp.float32),
            pltpu.SemaphoreType.DMA((_NBT,)),
            pltpu.SemaphoreType.DMA((2, _NDT)),
        ],
        compiler_params=pltpu.CompilerParams(
            use_tc_tiling_on_sc=False, needs_layout_passes=False),
    )
    def k(idx_hbm, table_hbm, out_hbm, idxb, rbuf, obuf, gsem, osem):
        wid = lax.axis_index("s") * _NC + lax.axis_index("c")
        b0 = wid * _BPW
        bt0 = wid * _NBT
        pltpu.sync_copy(idx_hbm.at[:, pl.ds(b0, _BPW)], idxb)

        iota = lax.iota(jnp.int32, 16)
        rows = [iota + 16 * kk for kk in range(8)]

        def gather_start(h, bt):
            pltpu.async_copy(
                table_hbm.at[idxb.at[h, pl.ds(bt * 128, 128)]],
                rbuf.at[bt], gsem.at[bt])

        def gather_wait(bt):
            pltpu.make_async_copy(
                table_hbm.at[idxb.at[0, pl.ds(0, 128)]],
                rbuf.at[bt], gsem.at[bt]).wait()

        def out_start(h, ob, dt):
            pltpu.async_copy(obuf.at[ob, dt],
                             out_hbm.at[h, dt, pl.ds(bt0, _NBT)],
                             osem.at[ob, dt])

        def out_wait(ob, dt):
            pltpu.make_async_copy(obuf.at[0, dt],
                                  out_hbm.at[0, dt, pl.ds(0, _NBT)],
                                  osem.at[ob, dt]).wait()

        for bt in range(_NBT):
            gather_start(0, bt)

        @pl.loop(0, HIST)
        def _(h):
            ob = lax.rem(h, 2)

            @pl.when(h >= 2)
            def _():
                for dt in range(_NDT):
                    out_wait(ob, dt)

            for bt in range(_NBT):
                gather_wait(bt)
                btv = iota * 0 + bt

                @pl.loop(0, _NDT)
                def _(dt):
                    for s in range(8):
                        colv = iota * 0 + (dt * 8 + s)
                        for kk in range(8):
                            v = plsc.load_gather(
                                rbuf, [btv, rows[kk], colv])
                            obuf[ob, dt, bt, s, pl.ds(16 * kk, 16)] = v

                @pl.when(h < HIST - 1)
                def _():
                    gather_start(h + 1, bt)

            for dt in range(_NDT):
                out_start(h, ob, dt)

        for ob in range(2):
            for dt in range(_NDT):
                out_wait(ob, dt)

    return k(idx_t, table)


def kernel(action, item_emb):
    idx_t = jnp.swapaxes(action, 0, 1).astype(jnp.int32)
    out5 = _sc_gather(idx_t, item_emb)
    return out5.transpose((2, 4, 0, 1, 3)).reshape(BATCH, HIST, EMBED_DIM)


# speedup vs baseline: 1.7031x; 1.7031x over previous
import functools

import jax
import jax.numpy as jnp
from jax import lax
from jax.experimental import pallas as pl
from jax.experimental.pallas import tpu as pltpu
from jax.experimental.pallas import tpu_sc as plsc

NUM_ITEMS = 1000000
EMBED_DIM = 64
BATCH = 16384
HIST = 50

_INFO = plsc.get_sparse_core_info()
_NC, _NS = _INFO.num_cores, _INFO.num_subcores
_NW = _NC * _NS
_BPW = BATCH // _NW
_NBT = _BPW // 128
_NDT = EMBED_DIM // 8


def _sc_gather(idx_t, table):
    mesh = plsc.VectorSubcoreMesh(core_axis_name="c", subcore_axis_name="s")

    @functools.partial(
        pl.kernel,
        out_type=jax.ShapeDtypeStruct(
            (HIST, _NDT, BATCH // 128, 8, 128), jnp.float32),
        mesh=mesh,
        scratch_types=[
            pltpu.VMEM((2, _BPW), jnp.int32),
            pltpu.VMEM((_NBT, 128, EMBED_DIM), jnp.float32),
            pltpu.VMEM((2, _NDT, _NBT, 8, 128), jnp.float32),
            pltpu.SemaphoreType.DMA((_NBT,)),
            pltpu.SemaphoreType.DMA((2, _NDT)),
            pltpu.SemaphoreType.DMA((2,)),
        ],
        compiler_params=pltpu.CompilerParams(
            use_tc_tiling_on_sc=False, needs_layout_passes=False),
    )
    def k(idx_hbm, table_hbm, out_hbm, ibuf, rbuf, obuf, gsem, osem, isem):
        wid = lax.axis_index("s") * _NC + lax.axis_index("c")
        b0 = wid * _BPW

        def stage_start(h, p):
            pltpu.async_copy(idx_hbm.at[h, pl.ds(b0, _BPW)], ibuf.at[p],
                             isem.at[p])

        def stage_wait(p):
            pltpu.make_async_copy(idx_hbm.at[0, pl.ds(0, _BPW)], ibuf.at[p],
                                  isem.at[p]).wait()

        iota = lax.iota(jnp.int32, 16)
        ldcol = []
        dthalf = []
        slow = []
        for d in range(16):
            c = (iota + d) & 15
            ldcol.append(c)
            dthalf.append(c >> 3)
            slow.append(c & 7)

        def gather_start(p, bt):
            pltpu.async_copy(
                table_hbm.at[ibuf.at[p, pl.ds(bt * 128, 128)]],
                rbuf.at[bt], gsem.at[bt])

        def gather_wait(bt):
            pltpu.make_async_copy(
                table_hbm.at[ibuf.at[0, pl.ds(0, 128)]],
                rbuf.at[bt], gsem.at[bt]).wait()

        bt0 = wid * _NBT

        def out_start(h, ob, dt):
            pltpu.async_copy(obuf.at[ob, dt],
                             out_hbm.at[h, dt, pl.ds(bt0, _NBT)],
                             osem.at[ob, dt])

        def out_wait(ob, dt):
            pltpu.make_async_copy(obuf.at[0, 0],
                                  out_hbm.at[0, 0, pl.ds(0, _NBT)],
                                  osem.at[ob, dt]).wait()

        stage_start(0, 0)
        stage_wait(0)
        for bt in range(_NBT):
            gather_start(0, bt)
        stage_start(1, 1)

        @pl.loop(0, HIST)
        def _(h):
            ob = lax.rem(h, 2)

            @pl.when(h >= 2)
            def _():
                for dt in range(_NDT):
                    out_wait(ob, dt)

            @pl.when(h < HIST - 1)
            def _():
                stage_wait(1 - ob)

            obv = iota * 0 + ob
            for bt in range(_NBT):
                gather_wait(bt)
                btv = iota * 0 + bt

                @pl.loop(0, 8)
                def _(m):
                    rowv = iota + m * 16

                    @pl.loop(0, 4)
                    def _(n):
                        coln = 16 * n
                        for d in range(16):
                            v = plsc.load_gather(
                                rbuf, [btv, rowv, ldcol[d] + coln])
                            plsc.store_scatter(
                                obuf,
                                [obv, dthalf[d] + 2 * n, btv, slow[d], rowv],
                                v)

                @pl.when(h < HIST - 1)
                def _():
                    gather_start(1 - ob, bt)

            @pl.when(h < HIST - 2)
            def _():
                stage_start(h + 2, ob)

            for dt in range(_NDT):
                out_start(h, ob, dt)

        for ob in range(2):
            for dt in range(_NDT):
                out_wait(ob, dt)

    return k(idx_t, table)


def kernel(action, item_emb):
    idx_t = jnp.swapaxes(action, 0, 1).astype(jnp.int32)
    out5 = _sc_gather(idx_t, item_emb)
    return out5.transpose((2, 4, 0, 1, 3)).reshape(BATCH, HIST, EMBED_DIM)


# speedup vs baseline: 1.9013x; 1.1164x over previous
import functools

import jax
import jax.numpy as jnp
from jax import lax
from jax.experimental import pallas as pl
from jax.experimental.pallas import tpu as pltpu
from jax.experimental.pallas import tpu_sc as plsc

NUM_ITEMS = 1000000
EMBED_DIM = 64
BATCH = 16384
HIST = 50

_INFO = plsc.get_sparse_core_info()
_NC, _NS = _INFO.num_cores, _INFO.num_subcores
_NW = _NC * _NS
_BPW = BATCH // _NW
_NG = _BPW // 64
_NBT = _BPW // 128
_NDT = EMBED_DIM // 8


def _sc_relayout(table_t, tail_lin):
    mesh = plsc.VectorSubcoreMesh(core_axis_name="c", subcore_axis_name="s")
    nfull_blocks = NUM_ITEMS // 128
    tpw = (nfull_blocks + _NW - 1) // _NW

    @functools.partial(
        pl.kernel,
        out_type=jax.ShapeDtypeStruct((NUM_ITEMS // 2, 128), jnp.float32),
        mesh=mesh,
        scratch_types=[
            pltpu.VMEM((2, 64, 128), jnp.float32),
            pltpu.VMEM((2, 64, 128), jnp.float32),
            pltpu.VMEM((32, 128), jnp.float32),
            pltpu.SemaphoreType.DMA((2,)),
            pltpu.SemaphoreType.DMA((2,)),
            pltpu.SemaphoreType.DMA((1,)),
        ],
        compiler_params=pltpu.CompilerParams(
            use_tc_tiling_on_sc=True, needs_layout_passes=False),
    )
    def k(tab_hbm, tail_hbm, lin_hbm, tbuf, obuf, tailbuf, isem, osem, tsem):
        wid = lax.axis_index("s") * _NC + lax.axis_index("c")
        iota = lax.iota(jnp.int32, 16)

        nfull = nfull_blocks

        def blk(t):
            return wid + _NW * t

        def in_start(t):
            ib = blk(t)
            s = lax.rem(t, 2)
            pltpu.async_copy(tab_hbm.at[:, pl.ds(ib * 128, 128)],
                             tbuf.at[s], isem.at[s])

        def in_wait(t):
            s = lax.rem(t, 2)
            pltpu.make_async_copy(tab_hbm.at[:, pl.ds(0, 128)],
                                  tbuf.at[s], isem.at[s]).wait()

        def out_start(t):
            ib = blk(t)
            s = lax.rem(t, 2)
            pltpu.async_copy(obuf.at[s], lin_hbm.at[pl.ds(ib * 64, 64)],
                             osem.at[s])

        def out_wait(t):
            s = lax.rem(t, 2)
            pltpu.make_async_copy(obuf.at[0], lin_hbm.at[pl.ds(0, 64)],
                                  osem.at[s]).wait()

        @pl.when(wid == 0)
        def _():
            pltpu.async_copy(tail_hbm, tailbuf, tsem.at[0])
            pltpu.make_async_copy(tail_hbm, tailbuf, tsem.at[0]).wait()
            pltpu.async_copy(tailbuf, lin_hbm.at[pl.ds(nfull * 64, 32)],
                             tsem.at[0])
            pltpu.make_async_copy(tailbuf, lin_hbm.at[pl.ds(0, 32)],
                                  tsem.at[0]).wait()

        @pl.when(blk(0) < nfull)
        def _():
            in_start(0)

        @pl.loop(0, tpw)
        def _(t):
            @pl.when(blk(t) < nfull)
            def _():
                @pl.when(blk(t + 1) < nfull)
                def _():
                    in_start(t + 1)
                in_wait(t)

                @pl.when(t >= 2)
                def _():
                    out_wait(t - 2)

                pv = iota * 0 + lax.rem(t, 2)

                @pl.loop(0, 8)
                def _(m):
                    itemv = 16 * m + iota
                    rowv = 8 * m + (iota >> 1)
                    colb = (iota & 1) * 64

                    @pl.loop(0, 4)
                    def _(n):
                        for dg in range(16):
                            cv = (iota + dg) & 15
                            v = plsc.load_gather(
                                tbuf, [pv, 16 * n + cv, itemv])
                            plsc.store_scatter(
                                obuf, [pv, rowv, colb + 16 * n + cv], v)

                out_start(t)

        for t in (tpw - 2, tpw - 1):
            @pl.when(blk(t) < nfull)
            def _():
                out_wait(t)

    return k(table_t, tail_lin)


def _sc_gather(idx2_t, off_t, table2):
    mesh = plsc.VectorSubcoreMesh(core_axis_name="c", subcore_axis_name="s")

    @functools.partial(
        pl.kernel,
        out_type=jax.ShapeDtypeStruct(
            (HIST, _NDT, BATCH // 128, 8, 128), jnp.float32),
        mesh=mesh,
        scratch_types=[
            pltpu.VMEM((2, _BPW), jnp.int32),
            pltpu.VMEM((2, _BPW), jnp.int32),
            pltpu.VMEM((4, 64, 128), jnp.float32),
            pltpu.VMEM((2, _NDT, _NBT, 8, 128), jnp.float32),
            pltpu.SemaphoreType.DMA((4,)),
            pltpu.SemaphoreType.DMA((2, _NDT)),
            pltpu.SemaphoreType.DMA((2, 2)),
        ],
        compiler_params=pltpu.CompilerParams(
            use_tc_tiling_on_sc=True, needs_layout_passes=False),
    )
    def k(idx_hbm, off_hbm, table_hbm, out_hbm,
          ibuf, fbuf, rbuf, obuf, gsem, osem, isem):
        wid = lax.axis_index("s") * _NC + lax.axis_index("c")
        b0 = wid * _BPW

        def stage_start(h, p):
            pltpu.async_copy(idx_hbm.at[h, pl.ds(b0, _BPW)], ibuf.at[p],
                             isem.at[p, 0])
            pltpu.async_copy(off_hbm.at[h, pl.ds(b0, _BPW)], fbuf.at[p],
                             isem.at[p, 1])

        def stage_wait(p):
            pltpu.make_async_copy(idx_hbm.at[0, pl.ds(0, _BPW)], ibuf.at[p],
                                  isem.at[p, 0]).wait()
            pltpu.make_async_copy(off_hbm.at[0, pl.ds(0, _BPW)], fbuf.at[p],
                                  isem.at[p, 1]).wait()

        iota = lax.iota(jnp.int32, 16)
        ldcol = []
        dthalf = []
        slow = []
        for d in range(16):
            c = (iota + d) & 15
            ldcol.append(c)
            dthalf.append(c >> 3)
            slow.append(c & 7)

        def gather_start(p, g):
            pltpu.async_copy(
                table_hbm.at[ibuf.at[p, pl.ds(g * 64, 64)]],
                rbuf.at[g % 4], gsem.at[g % 4])

        def gather_wait(g):
            pltpu.make_async_copy(
                table_hbm.at[ibuf.at[0, pl.ds(0, 64)]],
                rbuf.at[g % 4], gsem.at[g % 4]).wait()

        bt0 = wid * _NBT

        def out_start(h, ob, dt):
            pltpu.async_copy(obuf.at[ob, dt],
                             out_hbm.at[h, dt, pl.ds(bt0, _NBT)],
                             osem.at[ob, dt])

        def out_wait(ob, dt):
            pltpu.make_async_copy(obuf.at[0, 0],
                                  out_hbm.at[0, 0, pl.ds(0, _NBT)],
                                  osem.at[ob, dt]).wait()

        stage_start(0, 0)
        stage_wait(0)
        for g in range(4):
            gather_start(0, g)
        stage_start(1, 1)

        @pl.loop(0, HIST)
        def _(h):
            ob = lax.rem(h, 2)

            @pl.when(h >= 2)
            def _():
                for dt in range(_NDT):
                    out_wait(ob, dt)

            @pl.when(h < HIST - 1)
            def _():
                stage_wait(1 - ob)

            obv = iota * 0 + ob
            for g in range(_NG):
                gather_wait(g)
                sv = iota * 0 + (g % 4)
                btv = iota * 0 + (g // 2)
                rbase = (g & 1) * 64

                @pl.loop(0, 4)
                def _(m):
                    rowgv = iota + m * 16
                    rowv = rowgv + rbase
                    offv = plsc.load_gather(
                        fbuf, [obv, rowgv + g * 64])

                    @pl.loop(0, 4)
                    def _(n):
                        coln = 16 * n
                        for d in range(16):
                            v = plsc.load_gather(
                                rbuf, [sv, rowgv, ldcol[d] + coln + offv])
                            plsc.store_scatter(
                                obuf,
                                [obv, dthalf[d] + 2 * n, btv, slow[d], rowv],
                                v)

                if g < 4:
                    gather_start(ob, g + 4)
                else:
                    @pl.when(h < HIST - 1)
                    def _():
                        gather_start(1 - ob, g - 4)

            @pl.when(h < HIST - 2)
            def _():
                stage_start(h + 2, ob)

            for dt in range(_NDT):
                out_start(h, ob, dt)

        for ob in range(2):
            for dt in range(_NDT):
                out_wait(ob, dt)

    return k(idx2_t, off_t, table2)


def kernel(action, item_emb):
    idx_t = jnp.swapaxes(action, 0, 1).astype(jnp.int32)
    idx2_t = lax.shift_right_logical(idx_t, 1)
    off_t = (idx_t & 1) << 6
    table_t = jnp.swapaxes(item_emb, 0, 1)
    ntail0 = (NUM_ITEMS // 128) * 128
    tail_lin = item_emb[ntail0:].reshape(32, 2 * EMBED_DIM)
    table2 = _sc_relayout(table_t, tail_lin)
    out5 = _sc_gather(idx2_t, off_t, table2)
    return out5.transpose((2, 4, 0, 1, 3)).reshape(BATCH, HIST, EMBED_DIM)
